# Initial kernel scaffold; baseline (speedup 1.0000x reference)
import jax
import jax.numpy as jnp
from jax.experimental import pallas as pl


def kernel(x, adj_lists, W, b):
    raise NotImplementedError("write your pallas kernel here")


# speedup vs baseline: 3.2780x; 3.2780x over previous
import functools

import jax
import jax.numpy as jnp
from jax import lax
from jax.experimental import pallas as pl
from jax.experimental.pallas import tpu as pltpu
from jax.experimental.pallas import tpu_sc as plsc

N, D, K, O = 10000, 128, 32, 128
NC, NS, L = 2, 16, 16
NW = NC * NS
C = 4
CK = C * K
NG = (N + NW * C - 1) // (NW * C)
DV = D // L

_mesh = plsc.VectorSubcoreMesh(core_axis_name="c", subcore_axis_name="s")


@functools.partial(
    pl.kernel,
    out_type=jax.ShapeDtypeStruct((N, D), jnp.float32),
    mesh=_mesh,
    scratch_types=[
        pltpu.VMEM((CK,), jnp.int32),
        pltpu.VMEM((CK, D), jnp.float32),
        pltpu.VMEM((C, D), jnp.float32),
        pltpu.SemaphoreType.DMA,
    ],
)
def _gather_sum(adj_hbm, x_hbm, out_hbm, idx_v, rows_v, acc_v, sem):
    w = lax.axis_index("s") * NC + lax.axis_index("c")

    def round_body(g, carry):
        base = (g * NW + w) * C

        @pl.when(base < N)
        def _():
            pltpu.sync_copy(adj_hbm.at[pl.ds(base * K, CK)], idx_v)
            pltpu.async_copy(x_hbm.at[idx_v], rows_v, sem).wait()
            for c in range(C):
                def kstep(kk, acc):
                    r = c * K + kk
                    return tuple(acc[d] + rows_v[r, pl.ds(d * L, L)]
                                 for d in range(DV))
                acc0 = tuple(rows_v[c * K, pl.ds(d * L, L)] for d in range(DV))
                accs = lax.fori_loop(1, K, kstep, acc0)
                for d in range(DV):
                    acc_v[c, pl.ds(d * L, L)] = accs[d]
            pltpu.sync_copy(acc_v, out_hbm.at[pl.ds(base, C)])

        return carry

    lax.fori_loop(0, NG, round_body, 0)


RB = 1000


def _linear_body(x_ref, agg_ref, w1_ref, w2_ref, b_ref, o_ref):
    h = (jnp.dot(x_ref[...], w1_ref[...], preferred_element_type=jnp.float32)
         + jnp.dot(agg_ref[...], w2_ref[...], preferred_element_type=jnp.float32)
         + b_ref[...])
    o_ref[...] = jnp.maximum(h, 0.0)


@jax.jit
def kernel(x, adj_lists, W, b):
    adj_flat = adj_lists.reshape(-1)
    agg_sum = _gather_sum(adj_flat, x)

    w1t = W[:, :D].T
    w2t = W[:, D:].T * (1.0 / K)
    b2 = b.reshape(1, O)

    out = pl.pallas_call(
        _linear_body,
        grid=(N // RB,),
        in_specs=[
            pl.BlockSpec((RB, D), lambda i: (i, 0)),
            pl.BlockSpec((RB, D), lambda i: (i, 0)),
            pl.BlockSpec((D, O), lambda i: (0, 0)),
            pl.BlockSpec((D, O), lambda i: (0, 0)),
            pl.BlockSpec((1, O), lambda i: (0, 0)),
        ],
        out_specs=pl.BlockSpec((RB, O), lambda i: (i, 0)),
        out_shape=jax.ShapeDtypeStruct((N, O), jnp.float32),
    )(x, agg_sum, w1t, w2t, b2)
    return out


# speedup vs baseline: 7.4359x; 2.2685x over previous
import functools

import jax
import jax.numpy as jnp
from jax import lax
from jax.experimental import pallas as pl
from jax.experimental.pallas import tpu as pltpu
from jax.experimental.pallas import tpu_sc as plsc

N, D, K, O = 10000, 128, 32, 128
NC, NS, L = 2, 16, 16
NW = NC * NS
C = 4
CK = C * K
NCHUNK = N // C
NG = (NCHUNK + NW - 1) // NW
DV = D // L
NBUF = 4

_mesh = plsc.VectorSubcoreMesh(core_axis_name="c", subcore_axis_name="s")


@functools.partial(
    pl.kernel,
    out_type=jax.ShapeDtypeStruct((N, D), jnp.float32),
    mesh=_mesh,
    scratch_types=[
        pltpu.VMEM((NG * CK,), jnp.int32),
        pltpu.VMEM((NBUF, CK, D), jnp.float32),
        pltpu.VMEM((NBUF, C, D), jnp.float32),
        pltpu.SemaphoreType.DMA,
        pltpu.SemaphoreType.DMA,
        pltpu.SemaphoreType.DMA,
        pltpu.SemaphoreType.DMA,
        pltpu.SemaphoreType.DMA,
        pltpu.SemaphoreType.DMA,
        pltpu.SemaphoreType.DMA,
        pltpu.SemaphoreType.DMA,
    ],
)
def _gather_sum(adj_hbm, x_hbm, out_hbm, idx_all, rows_v, acc_v,
                g0, g1, g2, g3, s0, s1, s2, s3):
    gsem = (g0, g1, g2, g3)
    ssem = (s0, s1, s2, s3)
    w = lax.axis_index("s") * NC + lax.axis_index("c")
    ngw = (NCHUNK - w + NW - 1) // NW

    pltpu.sync_copy(adj_hbm.at[pl.ds(w * NG * CK, NG * CK)], idx_all)

    def base_of(gg):
        return (gg * NW + w) * C

    def gather(gg, b):
        return pltpu.make_async_copy(
            x_hbm.at[idx_all.at[pl.ds(gg * CK, CK)]], rows_v.at[b], gsem[b])

    def store(gg, b):
        return pltpu.make_async_copy(
            acc_v.at[b], out_hbm.at[pl.ds(base_of(gg), C)], ssem[b])

    for b in range(NBUF - 1):
        gather(b, b).start()

    def chunk_body(gg, b):
        pre = gg + NBUF - 1

        @pl.when(pre < ngw)
        def _():
            gather(pre, (b + NBUF - 1) % NBUF).start()

        @pl.when(gg >= NBUF)
        def _():
            store(gg - NBUF, b).wait()

        gather(gg, b).wait()

        for c in range(C):
            def kstep(kk, acc):
                a = acc
                for u in range(4):
                    r = c * K + kk * 4 + u
                    a = tuple(a[d] + rows_v[b, r, pl.ds(d * L, L)]
                              for d in range(DV))
                return a
            acc0 = tuple(jnp.zeros((L,), jnp.float32) for _ in range(DV))
            accs = lax.fori_loop(0, K // 4, kstep, acc0)
            for d in range(DV):
                acc_v[b, c, pl.ds(d * L, L)] = accs[d]

        store(gg, b).start()

    def quad_body(i, carry):
        for b in range(NBUF):
            gg = i * NBUF + b

            @pl.when(gg < ngw)
            def _():
                chunk_body(gg, b)
        return carry

    lax.fori_loop(0, (NG + NBUF - 1) // NBUF, quad_body, 0)

    for b in range(NBUF):
        gl = ngw - NBUF + jnp.remainder(b - (ngw - NBUF), NBUF)
        store(gl, b).wait()


RB = 1000


def _linear_body(x_ref, agg_ref, w1_ref, w2_ref, b_ref, o_ref):
    h = (jnp.dot(x_ref[...], w1_ref[...], preferred_element_type=jnp.float32)
         + jnp.dot(agg_ref[...], w2_ref[...], preferred_element_type=jnp.float32)
         + b_ref[...])
    o_ref[...] = jnp.maximum(h, 0.0)


@jax.jit
def kernel(x, adj_lists, W, b):
    adj2 = adj_lists.reshape(NCHUNK, CK)
    pad = jnp.zeros((NG * NW - NCHUNK, CK), jnp.int32)
    adj3 = (jnp.concatenate([adj2, pad])
            .reshape(NG, NW, CK).transpose(1, 0, 2).reshape(-1))

    agg_sum = _gather_sum(adj3, x)

    w1t = W[:, :D].T
    w2t = W[:, D:].T * (1.0 / K)
    b2 = b.reshape(1, O)

    out = pl.pallas_call(
        _linear_body,
        grid=(N // RB,),
        in_specs=[
            pl.BlockSpec((RB, D), lambda i: (i, 0)),
            pl.BlockSpec((RB, D), lambda i: (i, 0)),
            pl.BlockSpec((D, O), lambda i: (0, 0)),
            pl.BlockSpec((D, O), lambda i: (0, 0)),
            pl.BlockSpec((1, O), lambda i: (0, 0)),
        ],
        out_specs=pl.BlockSpec((RB, O), lambda i: (i, 0)),
        out_shape=jax.ShapeDtypeStruct((N, O), jnp.float32),
    )(x, agg_sum, w1t, w2t, b2)
    return out


# speedup vs baseline: 7.6981x; 1.0353x over previous
import functools

import jax
import jax.numpy as jnp
from jax import lax
from jax.experimental import pallas as pl
from jax.experimental.pallas import tpu as pltpu
from jax.experimental.pallas import tpu_sc as plsc

N, D, K, O = 10000, 128, 32, 128
NC, NS, L = 2, 16, 16
NW = NC * NS
C = 4
CK = C * K
NCHUNK = N // C
NG = (NCHUNK + NW - 1) // NW
DV = D // L
NBUF = 4

_mesh = plsc.VectorSubcoreMesh(core_axis_name="c", subcore_axis_name="s")


@functools.partial(
    pl.kernel,
    out_type=jax.ShapeDtypeStruct((N, D), jnp.float32),
    mesh=_mesh,
    scratch_types=[
        pltpu.VMEM((NBUF, CK), jnp.int32),
        pltpu.VMEM((NBUF, CK, D), jnp.float32),
        pltpu.VMEM((NBUF, C, D), jnp.float32),
        pltpu.SemaphoreType.DMA,
        pltpu.SemaphoreType.DMA,
        pltpu.SemaphoreType.DMA,
        pltpu.SemaphoreType.DMA,
        pltpu.SemaphoreType.DMA,
        pltpu.SemaphoreType.DMA,
        pltpu.SemaphoreType.DMA,
        pltpu.SemaphoreType.DMA,
        pltpu.SemaphoreType.DMA,
        pltpu.SemaphoreType.DMA,
        pltpu.SemaphoreType.DMA,
        pltpu.SemaphoreType.DMA,
    ],
)
def _gather_sum(adj_hbm, x_hbm, out_hbm, idx_v, rows_v, acc_v,
                i0, i1, i2, i3, g0, g1, g2, g3, s0, s1, s2, s3):
    isem = (i0, i1, i2, i3)
    gsem = (g0, g1, g2, g3)
    ssem = (s0, s1, s2, s3)
    w = lax.axis_index("s") * NC + lax.axis_index("c")
    ngw = (NCHUNK - w + NW - 1) // NW

    def base_of(gg):
        return (gg * NW + w) * C

    def fetch_idx(gg, b):
        return pltpu.make_async_copy(
            adj_hbm.at[pl.ds(base_of(gg) * K, CK)], idx_v.at[b], isem[b])

    def gather(gg, b):
        return pltpu.make_async_copy(
            x_hbm.at[idx_v.at[b]], rows_v.at[b], gsem[b])

    def store(gg, b):
        return pltpu.make_async_copy(
            acc_v.at[b], out_hbm.at[pl.ds(base_of(gg), C)], ssem[b])

    for b in range(NBUF):
        fetch_idx(b, b).start()
    for b in range(NBUF - 1):
        fetch_idx(b, b).wait()
        gather(b, b).start()

    def chunk_body(gg, b):
        gather(gg, b).wait()

        @pl.when(gg + NBUF < ngw)
        def _():
            fetch_idx(gg + NBUF, b).start()

        pre = gg + NBUF - 1
        bpre = (b + NBUF - 1) % NBUF

        @pl.when(pre < ngw)
        def _():
            fetch_idx(pre, bpre).wait()
            gather(pre, bpre).start()

        @pl.when(gg >= NBUF)
        def _():
            store(gg - NBUF, b).wait()

        for c in range(C):
            def kstep(kk, acc):
                a = acc
                for u in range(4):
                    r = c * K + kk * 4 + u
                    a = tuple(a[d] + rows_v[b, r, pl.ds(d * L, L)]
                              for d in range(DV))
                return a
            acc0 = tuple(jnp.zeros((L,), jnp.float32) for _ in range(DV))
            accs = lax.fori_loop(0, K // 4, kstep, acc0)
            for d in range(DV):
                acc_v[b, c, pl.ds(d * L, L)] = accs[d]

        store(gg, b).start()

    def quad_body(i, carry):
        for b in range(NBUF):
            gg = i * NBUF + b

            @pl.when(gg < ngw)
            def _():
                chunk_body(gg, b)
        return carry

    lax.fori_loop(0, (NG + NBUF - 1) // NBUF, quad_body, 0)

    for b in range(NBUF):
        gl = ngw - NBUF + jnp.remainder(b - (ngw - NBUF), NBUF)
        store(gl, b).wait()


RB = 1000
_DN = (((1,), (1,)), ((), ()))


def _linear_body(x_ref, agg_ref, w_ref, b_ref, o_ref):
    h = (lax.dot_general(x_ref[...], w_ref[:, :D], _DN,
                         preferred_element_type=jnp.float32)
         + lax.dot_general(agg_ref[...] * (1.0 / K), w_ref[:, D:], _DN,
                           preferred_element_type=jnp.float32)
         + b_ref[...])
    o_ref[...] = jnp.maximum(h, 0.0)


@jax.jit
def kernel(x, adj_lists, W, b):
    agg_sum = _gather_sum(adj_lists.reshape(-1), x)
    out = pl.pallas_call(
        _linear_body,
        grid=(N // RB,),
        in_specs=[
            pl.BlockSpec((RB, D), lambda i: (i, 0)),
            pl.BlockSpec((RB, D), lambda i: (i, 0)),
            pl.BlockSpec((O, 2 * D), lambda i: (0, 0)),
            pl.BlockSpec((1, O), lambda i: (0, 0)),
        ],
        out_specs=pl.BlockSpec((RB, O), lambda i: (i, 0)),
        out_shape=jax.ShapeDtypeStruct((N, O), jnp.float32),
    )(x, agg_sum, W, b.reshape(1, O))
    return out
